# baseline (speedup 1.0000x reference)
import functools

import jax
import jax.numpy as jnp
from jax.experimental import pallas as pl

F32 = jnp.float32


def _conv_body(x_ref, w_ref, b_ref, o_ref, *, taps, m, relu):
    lhs = jnp.concatenate([x_ref[0, pl.ds(t, m), :] for t in taps], axis=1)
    acc = jnp.dot(lhs, w_ref[...], preferred_element_type=F32) + b_ref[...]
    if relu:
        acc = jnp.maximum(acc, 0.0)
    o_ref[0, :, :] = acc


def _conv_layer(x, weff, bias, *, taps, m, relu):
    B, R, Cin = x.shape
    K, Cout = weff.shape
    return pl.pallas_call(
        functools.partial(_conv_body, taps=taps, m=m, relu=relu),
        grid=(B,),
        in_specs=[
            pl.BlockSpec((1, R, Cin), lambda b: (b, 0, 0)),
            pl.BlockSpec((K, Cout), lambda b: (0, 0)),
            pl.BlockSpec((1, Cout), lambda b: (0, 0)),
        ],
        out_specs=pl.BlockSpec((1, m, Cout), lambda b: (b, 0, 0)),
        out_shape=jax.ShapeDtypeStruct((B, m, Cout), F32),
    )(x, weff, bias)


def _conv_shift_body(x_ref, w_ref, b_ref, o_ref, *, taps, m, grid_w, valid_w,
                     shift, relu):
    lhs = jnp.concatenate([x_ref[0, pl.ds(t, m), :] for t in taps], axis=1)
    acc = jnp.dot(lhs, w_ref[...], preferred_element_type=F32) + b_ref[...]
    if relu:
        acc = jnp.maximum(acc, 0.0)
    rows = jax.lax.broadcasted_iota(jnp.int32, (m, 1), 0)
    mask = (jax.lax.rem(rows, grid_w) < valid_w).astype(F32)
    o_ref[0, 0:shift, :] = jnp.zeros((shift, o_ref.shape[2]), F32)
    o_ref[0, pl.ds(shift, m), :] = acc * mask
    tail = o_ref.shape[1] - shift - m
    o_ref[0, pl.ds(shift + m, tail), :] = jnp.zeros((tail, o_ref.shape[2]), F32)


def _conv_shift_layer(x, weff, bias, *, taps, m, grid_w, valid_w, shift,
                      out_rows, relu):
    B, R, Cin = x.shape
    K, Cout = weff.shape
    return pl.pallas_call(
        functools.partial(_conv_shift_body, taps=taps, m=m, grid_w=grid_w,
                          valid_w=valid_w, shift=shift, relu=relu),
        grid=(B,),
        in_specs=[
            pl.BlockSpec((1, R, Cin), lambda b: (b, 0, 0)),
            pl.BlockSpec((K, Cout), lambda b: (0, 0)),
            pl.BlockSpec((1, Cout), lambda b: (0, 0)),
        ],
        out_specs=pl.BlockSpec((1, out_rows, Cout), lambda b: (b, 0, 0)),
        out_shape=jax.ShapeDtypeStruct((B, out_rows, Cout), F32),
    )(x, weff, bias)


def _vq_body(x_ref, w_ref, b_ref, cbt_ref, cb_ref, c2_ref, q_ref, loss_ref, *,
             taps, m, grid_w, valid_w, shift):
    lhs = jnp.concatenate([x_ref[0, pl.ds(t, m), :] for t in taps], axis=1)
    z = jnp.dot(lhs, w_ref[...], preferred_element_type=F32) + b_ref[...]
    cross = jnp.dot(z, cbt_ref[...], preferred_element_type=F32)
    dist = c2_ref[...] - 2.0 * cross
    dmin = jnp.min(dist, axis=1, keepdims=True)
    lanes = jax.lax.broadcasted_iota(jnp.int32, dist.shape, 1)
    idx = jnp.min(jnp.where(dist == dmin, lanes, 10_000), axis=1, keepdims=True)
    onehot = (lanes == idx).astype(F32)
    quant = jnp.dot(onehot, cb_ref[...], preferred_element_type=F32)
    rows = jax.lax.broadcasted_iota(jnp.int32, (m, 1), 0)
    mask = (jax.lax.rem(rows, grid_w) < valid_w).astype(F32)
    diff = (quant - z) * mask
    part = jnp.sum(diff * diff, axis=(0, 1), keepdims=True)
    q_ref[0, 0:shift, :] = jnp.zeros((shift, q_ref.shape[2]), F32)
    q_ref[0, pl.ds(shift, m), :] = quant * mask
    tail = q_ref.shape[1] - shift - m
    q_ref[0, pl.ds(shift + m, tail), :] = jnp.zeros((tail, q_ref.shape[2]), F32)
    @pl.when(pl.program_id(0) == 0)
    def _init():
        loss_ref[...] = jnp.zeros((1, 1), F32)
    loss_ref[...] += part


def _vq_layer(x, weff, bias, cbt, cb, c2, *, taps, m, grid_w, valid_w, shift,
              out_rows):
    B, R, Cin = x.shape
    K, D = cb.shape
    return pl.pallas_call(
        functools.partial(_vq_body, taps=taps, m=m, grid_w=grid_w,
                          valid_w=valid_w, shift=shift),
        grid=(B,),
        in_specs=[
            pl.BlockSpec((1, R, Cin), lambda b: (b, 0, 0)),
            pl.BlockSpec(weff.shape, lambda b: (0, 0)),
            pl.BlockSpec((1, D), lambda b: (0, 0)),
            pl.BlockSpec((D, K), lambda b: (0, 0)),
            pl.BlockSpec((K, D), lambda b: (0, 0)),
            pl.BlockSpec((1, K), lambda b: (0, 0)),
        ],
        out_specs=[
            pl.BlockSpec((1, out_rows, D), lambda b: (b, 0, 0)),
            pl.BlockSpec((1, 1), lambda b: (0, 0)),
        ],
        out_shape=[
            jax.ShapeDtypeStruct((B, out_rows, D), F32),
            jax.ShapeDtypeStruct((1, 1), F32),
        ],
    )(x, weff, bias, cbt, cb, c2)


def _s2d(x, ph):
    B, H, W, C = x.shape
    x = x.reshape(B, H // ph, ph, W // ph, ph, C)
    return jnp.transpose(x, (0, 1, 3, 2, 4, 5)).reshape(
        B, H // ph, W // ph, ph * ph * C)


def kernel(x, W1, b1, W2, b2, W3, b3, codebook, D1w, D1b, D2w, D2b, D3w, D3b):
    B = x.shape[0]
    D = codebook.shape[1]

    W1e = jnp.zeros((16, 32), F32)
    for i in range(2):
        for j in range(2):
            for p in range(2):
                for q in range(2):
                    W1e = W1e.at[(i * 2 + j) * 4 + p * 2 + q, :].set(
                        W1[:, 0, 2 * i + p, 2 * j + q])
    W2e = jnp.zeros((512, 64), F32)
    for i in range(2):
        for j in range(2):
            for p in range(2):
                for q in range(2):
                    t = (i * 2 + j) * 128 + (p * 2 + q) * 32
                    W2e = W2e.at[t:t + 32, :].set(W2[:, :, 2 * i + p, 2 * j + q].T)
    W3e = jnp.zeros((576, D), F32)
    for di in range(3):
        for dj in range(3):
            t = (di * 3 + dj) * 64
            W3e = W3e.at[t:t + 64, :].set(W3[:, :, di, dj].T)
    D1e = jnp.zeros((9 * D, 64), F32)
    for di in range(3):
        for dj in range(3):
            t = (di * 3 + dj) * D
            D1e = D1e.at[t:t + D, :].set(D1w[:, :, 2 - di, 2 - dj])
    D2e = jnp.zeros((576, 128), F32)
    for r in range(2):
        for s in range(2):
            for a in range(2):
                for b in range(2):
                    t = ((r + a) * 3 + (s + b)) * 64
                    ph = (r * 2 + s) * 32
                    D2e = D2e.at[t:t + 64, ph:ph + 32].set(
                        D2w[:, :, 3 - (r + 2 * a), 3 - (s + 2 * b)])
    D3e = jnp.zeros((9 * 32, 4), F32)
    for r in range(2):
        for s in range(2):
            for a in range(2):
                for b in range(2):
                    t = ((r + a) * 3 + (s + b)) * 32
                    D3e = D3e.at[t:t + 32, r * 2 + s].set(
                        D3w[:, 0, 3 - (r + 2 * a), 3 - (s + 2 * b)])
    b2e = jnp.tile(D2b, 4)[None, :]
    b3e = jnp.broadcast_to(D3b, (4,))[None, :]
    c2 = jnp.sum(codebook * codebook, axis=1)[None, :]

    xp = jnp.pad(x.reshape(B, 224, 224)[..., None], ((0, 0), (1, 1), (1, 1), (0, 0)))
    xs = _s2d(xp, 2).reshape(B, 113 * 113, 4)
    taps2 = lambda w: [i * w + j for i in range(2) for j in range(2)]
    taps3 = lambda w: [i * w + j for i in range(3) for j in range(3)]
    y1 = _conv_layer(xs, W1e, b1[None, :], taps=taps2(113), m=12655, relu=True)
    y1 = jnp.pad(y1, ((0, 0), (0, 1), (0, 0))).reshape(B, 112, 113, 32)[:, :, :112]
    y1p = jnp.pad(y1, ((0, 0), (1, 1), (1, 1), (0, 0)))
    y1s = _s2d(y1p, 2).reshape(B, 57 * 57, 128)
    y2 = _conv_layer(y1s, W2e, b2[None, :], taps=taps2(57), m=3191, relu=True)
    y2 = jnp.pad(y2, ((0, 0), (0, 1), (0, 0))).reshape(B, 56, 57, 64)[:, :, :56]
    y2p = jnp.pad(y2, ((0, 0), (1, 1), (1, 1), (0, 0))).reshape(B, 58 * 58, 64)

    q, loss_sum = _vq_layer(
        y2p, W3e, b3[None, :], codebook.T, codebook, c2,
        taps=taps3(58), m=3246, grid_w=58, valid_w=56, shift=59, out_rows=3364)
    vq_loss = loss_sum[0, 0] / (B * 56 * 56 * D)

    r1 = _conv_shift_layer(
        q, D1e, D1b[None, :], taps=taps3(58), m=3246, grid_w=58, valid_w=56,
        shift=59, out_rows=3364, relu=True)
    p5 = _conv_layer(r1, D2e, b2e, taps=taps3(58), m=3246, relu=True)
    p5 = jnp.pad(p5, ((0, 0), (0, 2), (0, 0))).reshape(B, 56, 58, 4, 32)[:, :, :56]
    r2 = p5.reshape(B, 56, 56, 2, 2, 32).transpose(0, 1, 3, 2, 4, 5).reshape(
        B, 112, 112, 32)
    r2p = jnp.pad(r2, ((0, 0), (1, 1), (1, 1), (0, 0))).reshape(B, 114 * 114, 32)
    o6 = _conv_layer(r2p, D3e, b3e, taps=taps3(114), m=12766, relu=False)
    o6 = jnp.pad(o6, ((0, 0), (0, 2), (0, 0))).reshape(B, 112, 114, 2, 2)[:, :, :112]
    recon = o6.transpose(0, 1, 3, 2, 4).reshape(B, 1, 224, 224)
    return recon, vq_loss


# speedup vs baseline: 2.5749x; 2.5749x over previous
import jax
import jax.numpy as jnp
from jax.experimental import pallas as pl
from jax.experimental.pallas import tpu as pltpu

F32 = jnp.float32
P = 58
NROW = P * P
M1 = 3305
M = 3246
SH = P + 1


def _row_mask(m):
    rows = jax.lax.broadcasted_iota(jnp.int32, (m, 1), 0)
    return (jax.lax.rem(rows, P) < P - 2).astype(F32)


def _mega_body(xs4_ref, w1_ref, b1_ref, w2_ref, b2_ref, w3_ref, b3_ref,
               cbt_ref, cb_ref, c2_ref, d1_ref, d1b_ref, d2_ref, d2b_ref,
               w6_ref, b6_ref, o_ref, loss_ref,
               y1s_ref, y2g_ref, q_ref, r1_ref, p5_ref):
    acc = jnp.zeros((M1, 128), F32)
    for hi in range(2):
        for hj in range(2):
            t = hi * P + hj
            acc += jnp.dot(xs4_ref[0, pl.ds(t, M1), :],
                           w1_ref[(hi * 2 + hj) * 16:(hi * 2 + hj) * 16 + 16, :],
                           preferred_element_type=F32)
    acc = jnp.maximum(acc + b1_ref[...], 0.0)
    rows = jax.lax.broadcasted_iota(jnp.int32, (M1, 1), 0)
    u = rows // P
    v = jax.lax.rem(rows, P)
    lanes = jax.lax.broadcasted_iota(jnp.int32, (1, 128), 1)
    p = lanes // 64
    q = jax.lax.rem(lanes // 32, 2)
    bad = (((u == 0) & (p == 0)) | ((u == 56) & (p == 1))
           | ((v == 0) & (q == 0)) | ((v == 56) & (q == 1)) | (v == 57))
    y1s_ref[pl.ds(0, M1), :] = jnp.where(bad, 0.0, acc)
    y1s_ref[pl.ds(M1, NROW - M1), :] = jnp.zeros((NROW - M1, 128), F32)

    acc = jnp.zeros((M, 64), F32)
    for i in range(2):
        for j in range(2):
            t = i * P + j
            acc += jnp.dot(y1s_ref[pl.ds(t, M), :],
                           w2_ref[(i * 2 + j) * 128:(i * 2 + j) * 128 + 128, :],
                           preferred_element_type=F32)
    acc = jnp.maximum(acc + b2_ref[...], 0.0)
    mask = _row_mask(M)
    y2g_ref[pl.ds(0, SH), :] = jnp.zeros((SH, 64), F32)
    y2g_ref[pl.ds(SH, M), :] = acc * mask
    y2g_ref[pl.ds(SH + M, NROW - SH - M), :] = jnp.zeros((NROW - SH - M, 64), F32)

    z = jnp.zeros((M, 16), F32)
    for di in range(3):
        for dj in range(3):
            t = di * P + dj
            z += jnp.dot(y2g_ref[pl.ds(t, M), :],
                         w3_ref[(di * 3 + dj) * 64:(di * 3 + dj) * 64 + 64, :],
                         preferred_element_type=F32)
    z += b3_ref[...]
    cross = jnp.dot(z, cbt_ref[...], preferred_element_type=F32)
    dist = c2_ref[...] - 2.0 * cross
    dmin = jnp.min(dist, axis=1, keepdims=True)
    klanes = jax.lax.broadcasted_iota(jnp.int32, dist.shape, 1)
    idx = jnp.min(jnp.where(dist == dmin, klanes, 10_000), axis=1, keepdims=True)
    onehot = (klanes == idx).astype(F32)
    quant = jnp.dot(onehot, cb_ref[...], preferred_element_type=F32)
    diff = (quant - z) * mask
    part = jnp.sum(diff * diff, axis=(0, 1), keepdims=True)
    q_ref[pl.ds(0, SH), :] = jnp.zeros((SH, 16), F32)
    q_ref[pl.ds(SH, M), :] = quant * mask
    q_ref[pl.ds(SH + M, NROW - SH - M), :] = jnp.zeros((NROW - SH - M, 16), F32)
    @pl.when(pl.program_id(0) == 0)
    def _init():
        loss_ref[...] = jnp.zeros((1, 1), F32)
    loss_ref[...] += part

    acc = jnp.zeros((M, 64), F32)
    for di in range(3):
        for dj in range(3):
            t = di * P + dj
            acc += jnp.dot(q_ref[pl.ds(t, M), :],
                           d1_ref[(di * 3 + dj) * 16:(di * 3 + dj) * 16 + 16, :],
                           preferred_element_type=F32)
    acc = jnp.maximum(acc + d1b_ref[...], 0.0)
    r1_ref[pl.ds(0, SH), :] = jnp.zeros((SH, 64), F32)
    r1_ref[pl.ds(SH, M), :] = acc * mask
    r1_ref[pl.ds(SH + M, NROW - SH - M), :] = jnp.zeros((NROW - SH - M, 64), F32)

    acc = jnp.zeros((M, 128), F32)
    for di in range(3):
        for dj in range(3):
            t = di * P + dj
            acc += jnp.dot(r1_ref[pl.ds(t, M), :],
                           d2_ref[(di * 3 + dj) * 64:(di * 3 + dj) * 64 + 64, :],
                           preferred_element_type=F32)
    acc = jnp.maximum(acc + d2b_ref[...], 0.0)
    p5_ref[pl.ds(0, SH), :] = jnp.zeros((SH, 128), F32)
    p5_ref[pl.ds(SH, M), :] = acc * mask
    p5_ref[pl.ds(SH + M, NROW - SH - M), :] = jnp.zeros((NROW - SH - M, 128), F32)

    acc = jnp.zeros((M, 16), F32)
    for di in range(3):
        for dj in range(3):
            t = di * P + dj
            acc += jnp.dot(p5_ref[pl.ds(t, M), :],
                           w6_ref[(di * 3 + dj) * 128:(di * 3 + dj) * 128 + 128, :],
                           preferred_element_type=F32)
    o_ref[0, :, :] = acc + b6_ref[...]


def kernel(x, W1, b1, W2, b2, W3, b3, codebook, D1w, D1b, D2w, D2b, D3w, D3b):
    B = x.shape[0]
    D = codebook.shape[1]

    W1n = jnp.zeros((64, 128), F32)
    for hi in range(2):
        for hj in range(2):
            for tr in range(4):
                for tc in range(4):
                    for p_ in range(2):
                        for q_ in range(2):
                            dy = 4 * hi + tr - 2 * p_ - 1
                            dx = 4 * hj + tc - 2 * q_ - 1
                            if 0 <= dy <= 3 and 0 <= dx <= 3:
                                W1n = W1n.at[
                                    (hi * 2 + hj) * 16 + tr * 4 + tc,
                                    (p_ * 2 + q_) * 32:(p_ * 2 + q_) * 32 + 32,
                                ].set(W1[:, 0, dy, dx])
    W2n = jnp.zeros((512, 64), F32)
    for i in range(2):
        for j in range(2):
            for p_ in range(2):
                for q_ in range(2):
                    r0 = (i * 2 + j) * 128 + (p_ * 2 + q_) * 32
                    W2n = W2n.at[r0:r0 + 32, :].set(W2[:, :, 2 * i + p_, 2 * j + q_].T)
    W3n = jnp.zeros((576, D), F32)
    for di in range(3):
        for dj in range(3):
            t = (di * 3 + dj) * 64
            W3n = W3n.at[t:t + 64, :].set(W3[:, :, di, dj].T)
    D1n = jnp.zeros((9 * D, 64), F32)
    for di in range(3):
        for dj in range(3):
            t = (di * 3 + dj) * D
            D1n = D1n.at[t:t + D, :].set(D1w[:, :, 2 - di, 2 - dj])
    D2n = jnp.zeros((576, 128), F32)
    for r in range(2):
        for s in range(2):
            for a in range(2):
                for b in range(2):
                    t = ((r + a) * 3 + (s + b)) * 64
                    ph = (r * 2 + s) * 32
                    D2n = D2n.at[t:t + 64, ph:ph + 32].set(
                        D2w[:, :, 3 - (r + 2 * a), 3 - (s + 2 * b)])
    W6n = jnp.zeros((9 * 128, 16), F32)
    for di in range(3):
        for dj in range(3):
            for lo in range(2):
                for lof in range(2):
                    for e in range(2):
                        for rp in range(2):
                            for f in range(2):
                                for sp in range(2):
                                    a = 2 * di + lo - e - rp - 1
                                    b = 2 * dj + lof - f - sp - 1
                                    if a in (0, 1) and b in (0, 1):
                                        W6n = W6n.at[
                                            (di * 3 + dj) * 128 + (lo * 2 + lof) * 32:
                                            (di * 3 + dj) * 128 + (lo * 2 + lof) * 32 + 32,
                                            e * 8 + rp * 4 + f * 2 + sp,
                                        ].set(D3w[:, 0, 3 - (rp + 2 * a), 3 - (sp + 2 * b)])
    b1n = jnp.tile(b1, 4)[None, :]
    d2bn = jnp.tile(D2b, 4)[None, :]
    b6n = jnp.broadcast_to(D3b, (16,))[None, :]
    c2 = jnp.sum(codebook * codebook, axis=1)[None, :]

    xg = jnp.pad(x.reshape(B, 224, 224), ((0, 0), (4, 4), (4, 4)))
    xs4 = xg.reshape(B, P, 4, P, 4).transpose(0, 1, 3, 2, 4).reshape(B, NROW, 16)

    full = lambda shp: pl.BlockSpec(shp, lambda bb: (0,) * len(shp))
    o6, loss_sum = pl.pallas_call(
        _mega_body,
        grid=(B,),
        in_specs=[
            pl.BlockSpec((1, NROW, 16), lambda bb: (bb, 0, 0)),
            full((64, 128)), full((1, 128)),
            full((512, 64)), full((1, 64)),
            full((576, D)), full((1, D)),
            full((D, 32)), full((32, D)), full((1, 32)),
            full((9 * D, 64)), full((1, 64)),
            full((576, 128)), full((1, 128)),
            full((9 * 128, 16)), full((1, 16)),
        ],
        out_specs=[
            pl.BlockSpec((1, M, 16), lambda bb: (bb, 0, 0)),
            pl.BlockSpec((1, 1), lambda bb: (0, 0)),
        ],
        out_shape=[
            jax.ShapeDtypeStruct((B, M, 16), F32),
            jax.ShapeDtypeStruct((1, 1), F32),
        ],
        scratch_shapes=[
            pltpu.VMEM((NROW, 128), F32),
            pltpu.VMEM((NROW, 64), F32),
            pltpu.VMEM((NROW, 16), F32),
            pltpu.VMEM((NROW, 64), F32),
            pltpu.VMEM((NROW, 128), F32),
        ],
    )(xs4, W1n, b1n, W2n, b2[None, :], W3n, b3[None, :],
      codebook.T, codebook, c2, D1n, D1b[None, :], D2n, d2bn, W6n, b6n)

    vq_loss = loss_sum[0, 0] / (B * 56 * 56 * D)
    o6 = jnp.pad(o6, ((0, 0), (0, 2), (0, 0))).reshape(B, 56, P, 16)[:, :, :56]
    o6 = o6.reshape(B, 56, 56, 2, 2, 2, 2)
    recon = o6.transpose(0, 1, 3, 4, 2, 5, 6).reshape(B, 1, 224, 224)
    return recon, vq_loss


# speedup vs baseline: 2.6929x; 1.0458x over previous
import jax
import jax.numpy as jnp
from jax.experimental import pallas as pl
from jax.experimental.pallas import tpu as pltpu

F32 = jnp.float32
P = 58
NROW = P * P
M1 = 3305
M = 3246
SH = P + 1


CH = 256


def _chunks(m):
    return [(off, min(CH, m - off)) for off in range(0, m, CH)]


def _tap_matmul(read, w_ref, kc, taps, off, ch):
    acc = None
    for ti, t in enumerate(taps):
        d = jnp.dot(read(t + off, ch), w_ref[ti * kc:(ti + 1) * kc, :],
                    preferred_element_type=F32)
        acc = d if acc is None else acc + d
    return acc


def _conv_store(read, w_ref, b_ref, kc, taps, m, dst_ref, n, *, relu,
                shift, masked):
    if shift:
        dst_ref[pl.ds(0, shift), :] = jnp.zeros((shift, n), F32)
    for off, ch in _chunks(m):
        acc = _tap_matmul(read, w_ref, kc, taps, off, ch) + b_ref[...]
        if relu:
            acc = jnp.maximum(acc, 0.0)
        if masked:
            rows = off + jax.lax.broadcasted_iota(jnp.int32, (ch, 1), 0)
            acc *= (jax.lax.rem(rows, P) < P - 2).astype(F32)
        dst_ref[pl.ds(shift + off, ch), :] = acc
    tail = dst_ref.shape[0] - shift - m
    if tail > 0:
        dst_ref[pl.ds(shift + m, tail), :] = jnp.zeros((tail, n), F32)


TAPS2 = tuple(i * P + j for i in range(2) for j in range(2))
TAPS3 = tuple(i * P + j for i in range(3) for j in range(3))


def _mega_body(xs4_ref, w1_ref, b1_ref, w2_ref, b2_ref, w3_ref, b3_ref,
               cbt_ref, cb_ref, c2_ref, d1_ref, d1b_ref, d2_ref, d2b_ref,
               w6_ref, b6_ref, o_ref, loss_ref,
               y1s_ref, y2g_ref, q_ref, r1_ref, p5_ref):
    lanes = jax.lax.broadcasted_iota(jnp.int32, (1, 128), 1)
    p = lanes // 64
    q = jax.lax.rem(lanes // 32, 2)
    for off, ch in _chunks(M1):
        acc = _tap_matmul(lambda t, c: xs4_ref[0, pl.ds(t, c), :],
                          w1_ref, 16, TAPS2, off, ch)
        acc = jnp.maximum(acc + b1_ref[...], 0.0)
        rows = off + jax.lax.broadcasted_iota(jnp.int32, (ch, 1), 0)
        u = rows // P
        v = jax.lax.rem(rows, P)
        bad = (((u == 0) & (p == 0)) | ((u == 56) & (p == 1))
               | ((v == 0) & (q == 0)) | ((v == 56) & (q == 1)) | (v == 57))
        y1s_ref[pl.ds(off, ch), :] = jnp.where(bad, 0.0, acc)
    y1s_ref[pl.ds(M1, NROW - M1), :] = jnp.zeros((NROW - M1, 128), F32)

    _conv_store(lambda t, c: y1s_ref[pl.ds(t, c), :], w2_ref, b2_ref, 128,
                TAPS2, M, y2g_ref, 64, relu=True, shift=SH, masked=True)

    q_ref[pl.ds(0, SH), :] = jnp.zeros((SH, 16), F32)
    parts = []
    for off, ch in _chunks(M):
        z = _tap_matmul(lambda t, c: y2g_ref[pl.ds(t, c), :],
                        w3_ref, 64, TAPS3, off, ch) + b3_ref[...]
        cross = jnp.dot(z, cbt_ref[...], preferred_element_type=F32)
        dist = c2_ref[...] - 2.0 * cross
        dmin = jnp.min(dist, axis=1, keepdims=True)
        klanes = jax.lax.broadcasted_iota(jnp.int32, dist.shape, 1)
        idx = jnp.min(jnp.where(dist == dmin, klanes, 10_000),
                      axis=1, keepdims=True)
        onehot = (klanes == idx).astype(F32)
        quant = jnp.dot(onehot, cb_ref[...], preferred_element_type=F32)
        rows = off + jax.lax.broadcasted_iota(jnp.int32, (ch, 1), 0)
        mask = (jax.lax.rem(rows, P) < P - 2).astype(F32)
        diff = (quant - z) * mask
        parts.append(jnp.sum(diff * diff, axis=(0, 1), keepdims=True))
        q_ref[pl.ds(SH + off, ch), :] = quant * mask
    q_ref[pl.ds(SH + M, NROW - SH - M), :] = jnp.zeros((NROW - SH - M, 16), F32)
    part = sum(parts[1:], parts[0])
    @pl.when(pl.program_id(0) == 0)
    def _init():
        loss_ref[...] = jnp.zeros((1, 1), F32)
    loss_ref[...] += part

    _conv_store(lambda t, c: q_ref[pl.ds(t, c), :], d1_ref, d1b_ref, 16,
                TAPS3, M, r1_ref, 64, relu=True, shift=SH, masked=True)

    _conv_store(lambda t, c: r1_ref[pl.ds(t, c), :], d2_ref, d2b_ref, 64,
                TAPS3, M, p5_ref, 128, relu=True, shift=SH, masked=True)

    for off, ch in _chunks(M):
        acc = _tap_matmul(lambda t, c: p5_ref[pl.ds(t, c), :],
                          w6_ref, 128, TAPS3, off, ch)
        o_ref[0, pl.ds(off, ch), :] = acc + b6_ref[...]


def kernel(x, W1, b1, W2, b2, W3, b3, codebook, D1w, D1b, D2w, D2b, D3w, D3b):
    B = x.shape[0]
    D = codebook.shape[1]

    W1n = jnp.zeros((64, 128), F32)
    for hi in range(2):
        for hj in range(2):
            for tr in range(4):
                for tc in range(4):
                    for p_ in range(2):
                        for q_ in range(2):
                            dy = 4 * hi + tr - 2 * p_ - 1
                            dx = 4 * hj + tc - 2 * q_ - 1
                            if 0 <= dy <= 3 and 0 <= dx <= 3:
                                W1n = W1n.at[
                                    (hi * 2 + hj) * 16 + tr * 4 + tc,
                                    (p_ * 2 + q_) * 32:(p_ * 2 + q_) * 32 + 32,
                                ].set(W1[:, 0, dy, dx])
    W2n = jnp.zeros((512, 64), F32)
    for i in range(2):
        for j in range(2):
            for p_ in range(2):
                for q_ in range(2):
                    r0 = (i * 2 + j) * 128 + (p_ * 2 + q_) * 32
                    W2n = W2n.at[r0:r0 + 32, :].set(W2[:, :, 2 * i + p_, 2 * j + q_].T)
    W3n = jnp.zeros((576, D), F32)
    for di in range(3):
        for dj in range(3):
            t = (di * 3 + dj) * 64
            W3n = W3n.at[t:t + 64, :].set(W3[:, :, di, dj].T)
    D1n = jnp.zeros((9 * D, 64), F32)
    for di in range(3):
        for dj in range(3):
            t = (di * 3 + dj) * D
            D1n = D1n.at[t:t + D, :].set(D1w[:, :, 2 - di, 2 - dj])
    D2n = jnp.zeros((576, 128), F32)
    for r in range(2):
        for s in range(2):
            for a in range(2):
                for b in range(2):
                    t = ((r + a) * 3 + (s + b)) * 64
                    ph = (r * 2 + s) * 32
                    D2n = D2n.at[t:t + 64, ph:ph + 32].set(
                        D2w[:, :, 3 - (r + 2 * a), 3 - (s + 2 * b)])
    W6n = jnp.zeros((9 * 128, 16), F32)
    for di in range(3):
        for dj in range(3):
            for lo in range(2):
                for lof in range(2):
                    for e in range(2):
                        for rp in range(2):
                            for f in range(2):
                                for sp in range(2):
                                    a = 2 * di + lo - e - rp - 1
                                    b = 2 * dj + lof - f - sp - 1
                                    if a in (0, 1) and b in (0, 1):
                                        W6n = W6n.at[
                                            (di * 3 + dj) * 128 + (lo * 2 + lof) * 32:
                                            (di * 3 + dj) * 128 + (lo * 2 + lof) * 32 + 32,
                                            e * 8 + rp * 4 + f * 2 + sp,
                                        ].set(D3w[:, 0, 3 - (rp + 2 * a), 3 - (sp + 2 * b)])
    b1n = jnp.tile(b1, 4)[None, :]
    d2bn = jnp.tile(D2b, 4)[None, :]
    b6n = jnp.broadcast_to(D3b, (16,))[None, :]
    c2 = jnp.sum(codebook * codebook, axis=1)[None, :]

    xg = jnp.pad(x.reshape(B, 224, 224), ((0, 0), (4, 4), (4, 4)))
    xs4 = xg.reshape(B, P, 4, P, 4).transpose(0, 1, 3, 2, 4).reshape(B, NROW, 16)

    full = lambda shp: pl.BlockSpec(shp, lambda bb: (0,) * len(shp))
    o6, loss_sum = pl.pallas_call(
        _mega_body,
        grid=(B,),
        in_specs=[
            pl.BlockSpec((1, NROW, 16), lambda bb: (bb, 0, 0)),
            full((64, 128)), full((1, 128)),
            full((512, 64)), full((1, 64)),
            full((576, D)), full((1, D)),
            full((D, 32)), full((32, D)), full((1, 32)),
            full((9 * D, 64)), full((1, 64)),
            full((576, 128)), full((1, 128)),
            full((9 * 128, 16)), full((1, 16)),
        ],
        out_specs=[
            pl.BlockSpec((1, M, 16), lambda bb: (bb, 0, 0)),
            pl.BlockSpec((1, 1), lambda bb: (0, 0)),
        ],
        out_shape=[
            jax.ShapeDtypeStruct((B, M, 16), F32),
            jax.ShapeDtypeStruct((1, 1), F32),
        ],
        scratch_shapes=[
            pltpu.VMEM((NROW, 128), F32),
            pltpu.VMEM((NROW, 64), F32),
            pltpu.VMEM((NROW, 16), F32),
            pltpu.VMEM((NROW, 64), F32),
            pltpu.VMEM((NROW, 128), F32),
        ],
    )(xs4, W1n, b1n, W2n, b2[None, :], W3n, b3[None, :],
      codebook.T, codebook, c2, D1n, D1b[None, :], D2n, d2bn, W6n, b6n)

    vq_loss = loss_sum[0, 0] / (B * 56 * 56 * D)
    o6 = jnp.pad(o6, ((0, 0), (0, 2), (0, 0))).reshape(B, 56, P, 16)[:, :, :56]
    o6 = o6.reshape(B, 56, 56, 2, 2, 2, 2)
    recon = o6.transpose(0, 1, 3, 4, 2, 5, 6).reshape(B, 1, 224, 224)
    return recon, vq_loss


# speedup vs baseline: 2.7473x; 1.0202x over previous
import jax
import jax.numpy as jnp
from jax.experimental import pallas as pl
from jax.experimental.pallas import tpu as pltpu

F32 = jnp.float32
BF16 = jnp.bfloat16
P = 58
NROW = P * P
M1 = 3305
M = 3246
SH = P + 1


CH = 256


def _chunks(m):
    return [(off, min(CH, m - off)) for off in range(0, m, CH)]


def _tap_matmul(read, w_ref, kc, taps, off, ch):
    acc = None
    for ti, t in enumerate(taps):
        d = jnp.dot(read(t + off, ch), w_ref[ti * kc:(ti + 1) * kc, :],
                    preferred_element_type=F32)
        acc = d if acc is None else acc + d
    return acc


def _conv_store(read, w_ref, b_ref, kc, taps, m, dst_ref, n, *, relu,
                shift, masked):
    dt = dst_ref.dtype
    if shift:
        dst_ref[pl.ds(0, shift), :] = jnp.zeros((shift, n), dt)
    for off, ch in _chunks(m):
        acc = _tap_matmul(read, w_ref, kc, taps, off, ch) + b_ref[...]
        if relu:
            acc = jnp.maximum(acc, 0.0)
        if masked:
            rows = off + jax.lax.broadcasted_iota(jnp.int32, (ch, 1), 0)
            acc *= (jax.lax.rem(rows, P) < P - 2).astype(F32)
        dst_ref[pl.ds(shift + off, ch), :] = acc.astype(dt)
    tail = dst_ref.shape[0] - shift - m
    if tail > 0:
        dst_ref[pl.ds(shift + m, tail), :] = jnp.zeros((tail, n), dt)


TAPS2 = tuple(i * P + j for i in range(2) for j in range(2))
TAPS3 = tuple(i * P + j for i in range(3) for j in range(3))


def _mega_body(xs4_ref, w1_ref, b1_ref, w2_ref, b2_ref, w3_ref, b3_ref,
               cbt_ref, cb_ref, c2_ref, d1_ref, d1b_ref, d2_ref, d2b_ref,
               w6_ref, b6_ref, o_ref, loss_ref,
               y1s_ref, y2g_ref, q_ref, r1_ref, p5_ref):
    lanes = jax.lax.broadcasted_iota(jnp.int32, (1, 128), 1)
    p = lanes // 64
    q = jax.lax.rem(lanes // 32, 2)
    for off, ch in _chunks(M1):
        acc = _tap_matmul(lambda t, c: xs4_ref[0, pl.ds(t, c), :],
                          w1_ref, 16, TAPS2, off, ch)
        acc = jnp.maximum(acc + b1_ref[...], 0.0)
        rows = off + jax.lax.broadcasted_iota(jnp.int32, (ch, 1), 0)
        u = rows // P
        v = jax.lax.rem(rows, P)
        bad = (((u == 0) & (p == 0)) | ((u == 56) & (p == 1))
               | ((v == 0) & (q == 0)) | ((v == 56) & (q == 1)) | (v == 57))
        y1s_ref[pl.ds(off, ch), :] = jnp.where(bad, 0.0, acc)
    y1s_ref[pl.ds(M1, NROW - M1), :] = jnp.zeros((NROW - M1, 128), F32)

    _conv_store(lambda t, c: y1s_ref[pl.ds(t, c), :], w2_ref, b2_ref, 128,
                TAPS2, M, y2g_ref, 64, relu=True, shift=SH, masked=True)

    qdt = q_ref.dtype
    q_ref[pl.ds(0, SH), :] = jnp.zeros((SH, 16), qdt)
    parts = []
    for off, ch in _chunks(M):
        z = _tap_matmul(lambda t, c: y2g_ref[pl.ds(t, c), :],
                        w3_ref, 64, TAPS3, off, ch) + b3_ref[...]
        cross = jnp.dot(z, cbt_ref[...], preferred_element_type=F32)
        dist = c2_ref[...] - 2.0 * cross
        dmin = jnp.min(dist, axis=1, keepdims=True)
        klanes = jax.lax.broadcasted_iota(jnp.int32, dist.shape, 1)
        idx = jnp.min(jnp.where(dist == dmin, klanes, 10_000),
                      axis=1, keepdims=True)
        onehot = (klanes == idx).astype(F32)
        quant = jnp.dot(onehot, cb_ref[...], preferred_element_type=F32)
        rows = off + jax.lax.broadcasted_iota(jnp.int32, (ch, 1), 0)
        mask = (jax.lax.rem(rows, P) < P - 2).astype(F32)
        diff = (quant - z) * mask
        parts.append(jnp.sum(diff * diff, axis=(0, 1), keepdims=True))
        q_ref[pl.ds(SH + off, ch), :] = (quant * mask).astype(qdt)
    q_ref[pl.ds(SH + M, NROW - SH - M), :] = jnp.zeros((NROW - SH - M, 16), qdt)
    loss_ref[0, :, :] = sum(parts[1:], parts[0])

    _conv_store(lambda t, c: q_ref[pl.ds(t, c), :], d1_ref, d1b_ref, 16,
                TAPS3, M, r1_ref, 64, relu=True, shift=SH, masked=True)

    _conv_store(lambda t, c: r1_ref[pl.ds(t, c), :], d2_ref, d2b_ref, 64,
                TAPS3, M, p5_ref, 128, relu=True, shift=SH, masked=True)

    for off, ch in _chunks(M):
        acc = _tap_matmul(lambda t, c: p5_ref[pl.ds(t, c), :],
                          w6_ref, 128, TAPS3, off, ch)
        o_ref[0, pl.ds(off, ch), :] = acc + b6_ref[...]


def kernel(x, W1, b1, W2, b2, W3, b3, codebook, D1w, D1b, D2w, D2b, D3w, D3b):
    B = x.shape[0]
    D = codebook.shape[1]

    W1n = jnp.zeros((64, 128), F32)
    for hi in range(2):
        for hj in range(2):
            for tr in range(4):
                for tc in range(4):
                    for p_ in range(2):
                        for q_ in range(2):
                            dy = 4 * hi + tr - 2 * p_ - 1
                            dx = 4 * hj + tc - 2 * q_ - 1
                            if 0 <= dy <= 3 and 0 <= dx <= 3:
                                W1n = W1n.at[
                                    (hi * 2 + hj) * 16 + tr * 4 + tc,
                                    (p_ * 2 + q_) * 32:(p_ * 2 + q_) * 32 + 32,
                                ].set(W1[:, 0, dy, dx])
    W2n = jnp.zeros((512, 64), F32)
    for i in range(2):
        for j in range(2):
            for p_ in range(2):
                for q_ in range(2):
                    r0 = (i * 2 + j) * 128 + (p_ * 2 + q_) * 32
                    W2n = W2n.at[r0:r0 + 32, :].set(W2[:, :, 2 * i + p_, 2 * j + q_].T)
    W3n = jnp.zeros((576, D), F32)
    for di in range(3):
        for dj in range(3):
            t = (di * 3 + dj) * 64
            W3n = W3n.at[t:t + 64, :].set(W3[:, :, di, dj].T)
    D1n = jnp.zeros((9 * D, 64), F32)
    for di in range(3):
        for dj in range(3):
            t = (di * 3 + dj) * D
            D1n = D1n.at[t:t + D, :].set(D1w[:, :, 2 - di, 2 - dj])
    D2n = jnp.zeros((576, 128), F32)
    for r in range(2):
        for s in range(2):
            for a in range(2):
                for b in range(2):
                    t = ((r + a) * 3 + (s + b)) * 64
                    ph = (r * 2 + s) * 32
                    D2n = D2n.at[t:t + 64, ph:ph + 32].set(
                        D2w[:, :, 3 - (r + 2 * a), 3 - (s + 2 * b)])
    W6n = jnp.zeros((9 * 128, 16), F32)
    for di in range(3):
        for dj in range(3):
            for lo in range(2):
                for lof in range(2):
                    for e in range(2):
                        for rp in range(2):
                            for f in range(2):
                                for sp in range(2):
                                    a = 2 * di + lo - e - rp - 1
                                    b = 2 * dj + lof - f - sp - 1
                                    if a in (0, 1) and b in (0, 1):
                                        W6n = W6n.at[
                                            (di * 3 + dj) * 128 + (lo * 2 + lof) * 32:
                                            (di * 3 + dj) * 128 + (lo * 2 + lof) * 32 + 32,
                                            e * 8 + rp * 4 + f * 2 + sp,
                                        ].set(D3w[:, 0, 3 - (rp + 2 * a), 3 - (sp + 2 * b)])
    b1n = jnp.tile(b1, 4)[None, :]
    d2bn = jnp.tile(D2b, 4)[None, :]
    b6n = jnp.broadcast_to(D3b, (16,))[None, :]
    c2 = jnp.sum(codebook * codebook, axis=1)[None, :]

    xg = jnp.pad(x.reshape(B, 224, 224), ((0, 0), (4, 4), (4, 4)))
    xs4 = xg.reshape(B, P, 4, P, 4).transpose(0, 1, 3, 2, 4).reshape(B, NROW, 16)

    full = lambda shp: pl.BlockSpec(shp, lambda bb: (0,) * len(shp))
    o6, loss_sum = pl.pallas_call(
        _mega_body,
        grid=(B,),
        in_specs=[
            pl.BlockSpec((1, NROW, 16), lambda bb: (bb, 0, 0)),
            full((64, 128)), full((1, 128)),
            full((512, 64)), full((1, 64)),
            full((576, D)), full((1, D)),
            full((D, 32)), full((32, D)), full((1, 32)),
            full((9 * D, 64)), full((1, 64)),
            full((576, 128)), full((1, 128)),
            full((9 * 128, 16)), full((1, 16)),
        ],
        out_specs=[
            pl.BlockSpec((1, M, 16), lambda bb: (bb, 0, 0)),
            pl.BlockSpec((1, 1, 1), lambda bb: (bb, 0, 0)),
        ],
        out_shape=[
            jax.ShapeDtypeStruct((B, M, 16), F32),
            jax.ShapeDtypeStruct((B, 1, 1), F32),
        ],
        compiler_params=pltpu.CompilerParams(
            dimension_semantics=("parallel",)),
        scratch_shapes=[
            pltpu.VMEM((NROW, 128), F32),
            pltpu.VMEM((NROW, 64), F32),
            pltpu.VMEM((NROW, 16), BF16),
            pltpu.VMEM((NROW, 64), BF16),
            pltpu.VMEM((NROW, 128), BF16),
        ],
    )(xs4, W1n, b1n, W2n, b2[None, :], W3n, b3[None, :],
      codebook.T, codebook, c2, D1n.astype(BF16), D1b[None, :],
      D2n.astype(BF16), d2bn, W6n.astype(BF16), b6n)

    vq_loss = jnp.sum(loss_sum) / (B * 56 * 56 * D)
    o6 = jnp.pad(o6, ((0, 0), (0, 2), (0, 0))).reshape(B, 56, P, 16)[:, :, :56]
    o6 = o6.reshape(B, 56, 56, 2, 2, 2, 2)
    recon = o6.transpose(0, 1, 3, 4, 2, 5, 6).reshape(B, 1, 224, 224)
    return recon, vq_loss


# speedup vs baseline: 2.7684x; 1.0077x over previous
import jax
import jax.numpy as jnp
from jax.experimental import pallas as pl
from jax.experimental.pallas import tpu as pltpu

F32 = jnp.float32
BF16 = jnp.bfloat16
P = 58
NROW = P * P
M1 = 3305
M = 3246
SH = P + 1


CH = 256


def _chunks(m):
    return [(off, min(CH, m - off)) for off in range(0, m, CH)]


def _tap_matmul(read, w_ref, kc, taps, off, ch):
    acc = None
    for ti, t in enumerate(taps):
        d = jnp.dot(read(t + off, ch), w_ref[ti * kc:(ti + 1) * kc, :],
                    preferred_element_type=F32)
        acc = d if acc is None else acc + d
    return acc


def _conv_store(read, w_ref, b_ref, kc, taps, m, dst_ref, n, mask_ref, *,
                relu, shift):
    dt = dst_ref.dtype
    if shift:
        dst_ref[pl.ds(0, shift), :] = jnp.zeros((shift, n), dt)
    for off, ch in _chunks(m):
        acc = _tap_matmul(read, w_ref, kc, taps, off, ch) + b_ref[...]
        if relu:
            acc = jnp.maximum(acc, 0.0)
        if mask_ref is not None:
            acc *= mask_ref[pl.ds(off, ch), :]
        dst_ref[pl.ds(shift + off, ch), :] = acc.astype(dt)
    tail = dst_ref.shape[0] - shift - m
    if tail > 0:
        dst_ref[pl.ds(shift + m, tail), :] = jnp.zeros((tail, n), dt)


TAPS2 = tuple(i * P + j for i in range(2) for j in range(2))
TAPS3 = tuple(i * P + j for i in range(3) for j in range(3))


def _mega_body(xs4_ref, mask1_ref, mrow_ref, w1_ref, b1_ref, w2_ref, b2_ref,
               w3_ref, b3_ref, cbn_ref, cb_ref, c2_ref, d1_ref, d1b_ref,
               d2_ref, d2b_ref, w6_ref, b6_ref, o_ref, loss_ref,
               y1s_ref, y2g_ref, q_ref, r1_ref, p5_ref):
    for off, ch in _chunks(M1):
        acc = _tap_matmul(lambda t, c: xs4_ref[0, pl.ds(t, c), :],
                          w1_ref, 16, TAPS2, off, ch)
        acc = jnp.maximum(acc + b1_ref[...], 0.0)
        y1s_ref[pl.ds(off, ch), :] = acc * mask1_ref[pl.ds(off, ch), :]
    y1s_ref[pl.ds(M1, NROW - M1), :] = jnp.zeros((NROW - M1, 128), F32)

    _conv_store(lambda t, c: y1s_ref[pl.ds(t, c), :], w2_ref, b2_ref, 128,
                TAPS2, M, y2g_ref, 64, mrow_ref, relu=True, shift=SH)

    qdt = q_ref.dtype
    q_ref[pl.ds(0, SH), :] = jnp.zeros((SH, 16), qdt)
    parts = []
    for off, ch in _chunks(M):
        z = _tap_matmul(lambda t, c: y2g_ref[pl.ds(t, c), :],
                        w3_ref, 64, TAPS3, off, ch) + b3_ref[...]
        dist = jnp.dot(z, cbn_ref[...], preferred_element_type=F32) + c2_ref[...]
        dmin = jnp.min(dist, axis=1, keepdims=True)
        klanes = jax.lax.broadcasted_iota(jnp.int32, dist.shape, 1)
        idx = jnp.min(jnp.where(dist == dmin, klanes, 10_000),
                      axis=1, keepdims=True)
        onehot = (klanes == idx).astype(F32)
        quant = jnp.dot(onehot, cb_ref[...], preferred_element_type=F32)
        mask = mrow_ref[pl.ds(off, ch), :]
        diff = (quant - z) * mask
        parts.append(jnp.sum(diff * diff, axis=(0, 1), keepdims=True))
        q_ref[pl.ds(SH + off, ch), :] = (quant * mask).astype(qdt)
    q_ref[pl.ds(SH + M, NROW - SH - M), :] = jnp.zeros((NROW - SH - M, 16), qdt)
    loss_ref[0, :, :] = sum(parts[1:], parts[0])

    _conv_store(lambda t, c: q_ref[pl.ds(t, c), :], d1_ref, d1b_ref, 16,
                TAPS3, M, r1_ref, 64, mrow_ref, relu=True, shift=SH)

    _conv_store(lambda t, c: r1_ref[pl.ds(t, c), :], d2_ref, d2b_ref, 64,
                TAPS3, M, p5_ref, 128, mrow_ref, relu=True, shift=SH)

    for off, ch in _chunks(M):
        acc = _tap_matmul(lambda t, c: p5_ref[pl.ds(t, c), :],
                          w6_ref, 128, TAPS3, off, ch)
        o_ref[0, pl.ds(off, ch), :] = acc + b6_ref[...]


def kernel(x, W1, b1, W2, b2, W3, b3, codebook, D1w, D1b, D2w, D2b, D3w, D3b):
    B = x.shape[0]
    D = codebook.shape[1]

    W1n = jnp.zeros((64, 128), F32)
    for hi in range(2):
        for hj in range(2):
            for tr in range(4):
                for tc in range(4):
                    for p_ in range(2):
                        for q_ in range(2):
                            dy = 4 * hi + tr - 2 * p_ - 1
                            dx = 4 * hj + tc - 2 * q_ - 1
                            if 0 <= dy <= 3 and 0 <= dx <= 3:
                                W1n = W1n.at[
                                    (hi * 2 + hj) * 16 + tr * 4 + tc,
                                    (p_ * 2 + q_) * 32:(p_ * 2 + q_) * 32 + 32,
                                ].set(W1[:, 0, dy, dx])
    W2n = jnp.zeros((512, 64), F32)
    for i in range(2):
        for j in range(2):
            for p_ in range(2):
                for q_ in range(2):
                    r0 = (i * 2 + j) * 128 + (p_ * 2 + q_) * 32
                    W2n = W2n.at[r0:r0 + 32, :].set(W2[:, :, 2 * i + p_, 2 * j + q_].T)
    W3n = jnp.zeros((576, D), F32)
    for di in range(3):
        for dj in range(3):
            t = (di * 3 + dj) * 64
            W3n = W3n.at[t:t + 64, :].set(W3[:, :, di, dj].T)
    D1n = jnp.zeros((9 * D, 64), F32)
    for di in range(3):
        for dj in range(3):
            t = (di * 3 + dj) * D
            D1n = D1n.at[t:t + D, :].set(D1w[:, :, 2 - di, 2 - dj])
    D2n = jnp.zeros((576, 128), F32)
    for r in range(2):
        for s in range(2):
            for a in range(2):
                for b in range(2):
                    t = ((r + a) * 3 + (s + b)) * 64
                    ph = (r * 2 + s) * 32
                    D2n = D2n.at[t:t + 64, ph:ph + 32].set(
                        D2w[:, :, 3 - (r + 2 * a), 3 - (s + 2 * b)])
    W6n = jnp.zeros((9 * 128, 16), F32)
    for di in range(3):
        for dj in range(3):
            for lo in range(2):
                for lof in range(2):
                    for e in range(2):
                        for rp in range(2):
                            for f in range(2):
                                for sp in range(2):
                                    a = 2 * di + lo - e - rp - 1
                                    b = 2 * dj + lof - f - sp - 1
                                    if a in (0, 1) and b in (0, 1):
                                        W6n = W6n.at[
                                            (di * 3 + dj) * 128 + (lo * 2 + lof) * 32:
                                            (di * 3 + dj) * 128 + (lo * 2 + lof) * 32 + 32,
                                            e * 8 + rp * 4 + f * 2 + sp,
                                        ].set(D3w[:, 0, 3 - (rp + 2 * a), 3 - (sp + 2 * b)])
    b1n = jnp.tile(b1, 4)[None, :]
    d2bn = jnp.tile(D2b, 4)[None, :]
    b6n = jnp.broadcast_to(D3b, (16,))[None, :]
    c2 = jnp.sum(codebook * codebook, axis=1)[None, :]
    cbn = -2.0 * codebook.T
    rows1 = jnp.arange(M1)
    u = rows1 // P
    v = rows1 % P
    lanes = jnp.arange(128)
    pp = lanes // 64
    qq = (lanes // 32) % 2
    bad = (((u[:, None] == 0) & (pp[None, :] == 0))
           | ((u[:, None] == 56) & (pp[None, :] == 1))
           | ((v[:, None] == 0) & (qq[None, :] == 0))
           | ((v[:, None] == 56) & (qq[None, :] == 1))
           | (v[:, None] == 57))
    mask1 = jnp.where(bad, 0.0, 1.0).astype(F32)
    mrow = ((jnp.arange(M) % P) < P - 2).astype(F32)[:, None]

    xg = jnp.pad(x.reshape(B, 224, 224), ((0, 0), (4, 4), (4, 4)))
    xs4 = xg.reshape(B, P, 4, P, 4).transpose(0, 1, 3, 2, 4).reshape(B, NROW, 16)

    full = lambda shp: pl.BlockSpec(shp, lambda bb: (0,) * len(shp))
    o6, loss_sum = pl.pallas_call(
        _mega_body,
        grid=(B,),
        in_specs=[
            pl.BlockSpec((1, NROW, 16), lambda bb: (bb, 0, 0)),
            full((M1, 128)), full((M, 1)),
            full((64, 128)), full((1, 128)),
            full((512, 64)), full((1, 64)),
            full((576, D)), full((1, D)),
            full((D, 32)), full((32, D)), full((1, 32)),
            full((9 * D, 64)), full((1, 64)),
            full((576, 128)), full((1, 128)),
            full((9 * 128, 16)), full((1, 16)),
        ],
        out_specs=[
            pl.BlockSpec((1, M, 16), lambda bb: (bb, 0, 0)),
            pl.BlockSpec((1, 1, 1), lambda bb: (bb, 0, 0)),
        ],
        out_shape=[
            jax.ShapeDtypeStruct((B, M, 16), F32),
            jax.ShapeDtypeStruct((B, 1, 1), F32),
        ],
        compiler_params=pltpu.CompilerParams(
            dimension_semantics=("parallel",)),
        scratch_shapes=[
            pltpu.VMEM((NROW, 128), F32),
            pltpu.VMEM((NROW, 64), F32),
            pltpu.VMEM((NROW, 16), BF16),
            pltpu.VMEM((NROW, 64), BF16),
            pltpu.VMEM((NROW, 128), BF16),
        ],
    )(xs4, mask1, mrow, W1n, b1n, W2n, b2[None, :], W3n, b3[None, :],
      cbn, codebook, c2, D1n.astype(BF16), D1b[None, :],
      D2n.astype(BF16), d2bn, W6n.astype(BF16), b6n)

    vq_loss = jnp.sum(loss_sum) / (B * 56 * 56 * D)
    o6 = jnp.pad(o6, ((0, 0), (0, 2), (0, 0))).reshape(B, 56, P, 16)[:, :, :56]
    o6 = o6.reshape(B, 56, 56, 2, 2, 2, 2)
    recon = o6.transpose(0, 1, 3, 4, 2, 5, 6).reshape(B, 1, 224, 224)
    return recon, vq_loss
